# Initial kernel scaffold; baseline (speedup 1.0000x reference)
import jax
import jax.numpy as jnp
from jax.experimental import pallas as pl


def kernel(xyz, x, W1, b1, W2, b2, gamma, beta):
    raise NotImplementedError("write your pallas kernel here")


# speedup vs baseline: 10.6069x; 10.6069x over previous
import functools

import jax
import jax.numpy as jnp
from jax import lax
from jax.experimental import pallas as pl
from jax.experimental.pallas import tpu as pltpu
from jax.experimental.pallas import tpu_sc as plsc

RADIUS = 0.2
KNBR = 32
STRIDE = 4
EPS = 1e-5

_SUB = 8


def _fps_kernel(xyz_cl_ref, rows_ref, new_ref):
    CL = xyz_cl_ref.shape[2]
    N = CL * _SUB
    S = new_ref.shape[1]
    X = xyz_cl_ref[0, 0:_SUB, :]
    Y = xyz_cl_ref[0, _SUB:2 * _SUB, :]
    Z = xyz_cl_ref[0, 2 * _SUB:3 * _SUB, :]
    n_iota = (lax.broadcasted_iota(jnp.int32, (_SUB, CL), 0) * CL
              + lax.broadcasted_iota(jnp.int32, (_SUB, CL), 1))

    def body(i, carry):
        far, dists = carry
        row = rows_ref[0, pl.ds(far, 1), :]
        new_ref[0, pl.ds(i, 1), :] = row
        cx = row[:, 0:1]
        cy = row[:, 1:2]
        cz = row[:, 2:3]
        dx = X - cx
        dy = Y - cy
        dz = Z - cz
        d = (dx * dx + dy * dy) + dz * dz
        dists = jnp.minimum(dists, d)
        m = jnp.max(dists)
        nxt = jnp.min(jnp.where(dists == m, n_iota, N))
        return nxt.astype(jnp.int32), dists

    init = (jnp.int32(0), jnp.full((_SUB, CL), 1e10, dtype=jnp.float32))
    lax.fori_loop(0, S, body, init)


def _ballq_kernel(xyzT_ref, new_ref, idx_ref):
    b = pl.program_id(0)
    N = xyzT_ref.shape[2]
    QT = new_ref.shape[1]
    q = new_ref[0]
    dx = q[:, 0:1] - xyzT_ref[0, 0:1, :]
    dy = q[:, 1:2] - xyzT_ref[0, 1:2, :]
    dz = q[:, 2:3] - xyzT_ref[0, 2:3, :]
    d2 = (dx * dx + dy * dy) + dz * dz
    mask = (d2 < RADIUS * RADIUS).astype(jnp.int32)
    lane = lax.broadcasted_iota(jnp.int32, (QT, N), 1)
    c = mask
    sh = 1
    while sh < N:
        rolled = jnp.roll(c, sh, axis=1)
        c = c + jnp.where(lane >= sh, rolled, 0)
        sh *= 2
    cols = []
    cnt0 = jnp.sum((c <= 0).astype(jnp.int32), axis=1, keepdims=True)
    pad = jnp.where(cnt0 >= N, 0, cnt0)
    cols.append(pad)
    for j in range(1, KNBR):
        cj = jnp.sum((c <= j).astype(jnp.int32), axis=1, keepdims=True)
        cols.append(jnp.where(cj >= N, pad, cj))
    idx_ref[0] = jnp.concatenate(cols, axis=1) + b * N


def _dense_kernel(x_ref, xyzT_ref, w1f_ref, w1x_ref, b1_ref, z_ref):
    zf = lax.dot_general(x_ref[0], w1f_ref[...], (((0,), (1,)), ((), ())),
                         preferred_element_type=jnp.float32)
    zx = lax.dot_general(xyzT_ref[0], w1x_ref[...], (((0,), (1,)), ((), ())),
                         preferred_element_type=jnp.float32)
    z_ref[0] = (zf + zx) + b1_ref[...]


def _gather_rows(table, gidx):
    R = gidx.shape[0]
    D = table.shape[1]
    info = plsc.get_sparse_core_info()
    NC, NS = info.num_cores, info.num_subcores
    NW = NC * NS
    per_w = R // NW
    CH = 128
    steps = per_w // CH
    mesh = plsc.VectorSubcoreMesh(core_axis_name="c", subcore_axis_name="s")

    @functools.partial(
        pl.kernel,
        mesh=mesh,
        out_type=jax.ShapeDtypeStruct((R, D), jnp.float32),
        scratch_types=[
            pltpu.VMEM((CH,), jnp.int32),
            pltpu.VMEM((CH, D), jnp.float32),
            pltpu.SemaphoreType.DMA,
        ],
    )
    def gk(table_hbm, gidx_hbm, out_hbm, idx_v, rows_v, sem):
        wid = lax.axis_index("s") * NC + lax.axis_index("c")
        base = wid * per_w

        def step(i, carry):
            off = base + i * CH
            pltpu.sync_copy(gidx_hbm.at[pl.ds(off, CH)], idx_v)
            pltpu.async_copy(table_hbm.at[idx_v], rows_v, sem).wait()
            pltpu.sync_copy(rows_v, out_hbm.at[pl.ds(off, CH)])
            return carry

        lax.fori_loop(0, steps, step, 0)

    return gk(table, gidx)


def _mlp_kernel(g_ref, new_ref, w2_ref, b2_ref, w1x_ref, out_ref):
    QT, K, C1 = g_ref.shape
    C2 = out_ref.shape[1]
    q = new_ref[0]
    v = lax.dot_general(q, w1x_ref[...], (((1,), (1,)), ((), ())),
                        preferred_element_type=jnp.float32)
    h = jnp.maximum(g_ref[...] - v[:, None, :], 0.0)
    o = lax.dot_general(h.reshape(QT * K, C1), w2_ref[...],
                        (((1,), (1,)), ((), ())),
                        preferred_element_type=jnp.float32)
    o = o + b2_ref[...]
    out_ref[...] = jnp.max(o.reshape(QT, K, C2), axis=1)


def _bn_kernel(p_ref, gamma_ref, beta_ref, out_ref):
    p = p_ref[...]
    n = p.shape[0]
    mean = jnp.sum(p, axis=0, keepdims=True) / n
    d = p - mean
    var = jnp.sum(d * d, axis=0, keepdims=True) / n
    out_ref[...] = d / jnp.sqrt(var + EPS) * gamma_ref[...] + beta_ref[...]


def kernel(xyz, x, W1, b1, W2, b2, gamma, beta):
    B, N, _ = xyz.shape
    C0 = x.shape[1]
    C1 = W1.shape[0]
    C2 = W2.shape[0]
    S = N // STRIDE
    QT = 128
    QT2 = 128

    xyzT = jnp.transpose(xyz, (0, 2, 1))
    xyz_cl = xyzT.reshape(B, 3 * _SUB, N // _SUB)
    W1x = W1[:, :3]
    W1f = W1[:, 3:]
    b1r = b1.reshape(1, C1)
    b2r = b2.reshape(1, C2)
    gammar = gamma.reshape(1, C2)
    betar = beta.reshape(1, C2)

    new_xyz = pl.pallas_call(
        _fps_kernel,
        grid=(B,),
        in_specs=[
            pl.BlockSpec((1, 3 * _SUB, N // _SUB), lambda b: (b, 0, 0)),
            pl.BlockSpec((1, N, 3), lambda b: (b, 0, 0)),
        ],
        out_specs=pl.BlockSpec((1, S, 3), lambda b: (b, 0, 0)),
        out_shape=jax.ShapeDtypeStruct((B, S, 3), jnp.float32),
    )(xyz_cl, xyz)

    gidx = pl.pallas_call(
        _ballq_kernel,
        grid=(B, S // QT),
        in_specs=[
            pl.BlockSpec((1, 3, N), lambda b, t: (b, 0, 0)),
            pl.BlockSpec((1, QT, 3), lambda b, t: (b, t, 0)),
        ],
        out_specs=pl.BlockSpec((1, QT, KNBR), lambda b, t: (b, t, 0)),
        out_shape=jax.ShapeDtypeStruct((B, S, KNBR), jnp.int32),
    )(xyzT, new_xyz)

    z = pl.pallas_call(
        _dense_kernel,
        grid=(B,),
        in_specs=[
            pl.BlockSpec((1, C0, N), lambda b: (b, 0, 0)),
            pl.BlockSpec((1, 3, N), lambda b: (b, 0, 0)),
            pl.BlockSpec((C1, C0), lambda b: (0, 0)),
            pl.BlockSpec((C1, 3), lambda b: (0, 0)),
            pl.BlockSpec((1, C1), lambda b: (0, 0)),
        ],
        out_specs=pl.BlockSpec((1, N, C1), lambda b: (b, 0, 0)),
        out_shape=jax.ShapeDtypeStruct((B, N, C1), jnp.float32),
    )(x, xyzT, W1f, W1x, b1r)

    g = _gather_rows(z.reshape(B * N, C1), gidx.reshape(B * S * KNBR))
    g3 = g.reshape(B * S, KNBR, C1)

    nt = S // QT2
    pooled = pl.pallas_call(
        _mlp_kernel,
        grid=(B, nt),
        in_specs=[
            pl.BlockSpec((QT2, KNBR, C1), lambda b, t: (b * nt + t, 0, 0)),
            pl.BlockSpec((1, QT2, 3), lambda b, t: (b, t, 0)),
            pl.BlockSpec((C2, C1), lambda b, t: (0, 0)),
            pl.BlockSpec((1, C2), lambda b, t: (0, 0)),
            pl.BlockSpec((C1, 3), lambda b, t: (0, 0)),
        ],
        out_specs=pl.BlockSpec((QT2, C2), lambda b, t: (b * nt + t, 0)),
        out_shape=jax.ShapeDtypeStruct((B * S, C2), jnp.float32),
    )(g3, new_xyz, W2, b2r, W1x)

    bnout = pl.pallas_call(
        _bn_kernel,
        in_specs=[
            pl.BlockSpec((B * S, C2), lambda: (0, 0)),
            pl.BlockSpec((1, C2), lambda: (0, 0)),
            pl.BlockSpec((1, C2), lambda: (0, 0)),
        ],
        out_specs=pl.BlockSpec((B * S, C2), lambda: (0, 0)),
        out_shape=jax.ShapeDtypeStruct((B * S, C2), jnp.float32),
    )(pooled, gammar, betar)

    feat = jnp.transpose(bnout.reshape(B, S, C2), (0, 2, 1))
    return (new_xyz, feat)


# speedup vs baseline: 11.4705x; 1.0814x over previous
import functools

import jax
import jax.numpy as jnp
from jax import lax
from jax.experimental import pallas as pl
from jax.experimental.pallas import tpu as pltpu
from jax.experimental.pallas import tpu_sc as plsc

RADIUS = 0.2
KNBR = 32
STRIDE = 4
EPS = 1e-5

_SUB = 8


def _fps_kernel(xyz_cl_ref, rows_ref, new_ref):
    B = xyz_cl_ref.shape[0]
    CL = xyz_cl_ref.shape[2]
    N = CL * _SUB
    S = new_ref.shape[1]
    coords = []
    for b in range(B):
        coords.append((xyz_cl_ref[b, 0:_SUB, :],
                       xyz_cl_ref[b, _SUB:2 * _SUB, :],
                       xyz_cl_ref[b, 2 * _SUB:3 * _SUB, :]))
    n_iota = (lax.broadcasted_iota(jnp.int32, (_SUB, CL), 0) * CL
              + lax.broadcasted_iota(jnp.int32, (_SUB, CL), 1))

    def body(i, carry):
        out = []
        for b in range(B):
            far, dists = carry[b]
            X, Y, Z = coords[b]
            row = rows_ref[b, pl.ds(far, 1), :]
            new_ref[b, pl.ds(i, 1), :] = row
            cx = row[:, 0:1]
            cy = row[:, 1:2]
            cz = row[:, 2:3]
            dx = X - cx
            dy = Y - cy
            dz = Z - cz
            d = (dx * dx + dy * dy) + dz * dz
            dists = jnp.minimum(dists, d)
            m = jnp.max(dists)
            nxt = jnp.min(jnp.where(dists == m, n_iota, N))
            out.append((nxt.astype(jnp.int32), dists))
        return tuple(out)

    init = tuple((jnp.int32(0), jnp.full((_SUB, CL), 1e10, dtype=jnp.float32))
                 for _ in range(B))
    lax.fori_loop(0, S, body, init)


def _ballq_kernel(xyzT_ref, new_ref, idx_ref):
    b = pl.program_id(0)
    N = xyzT_ref.shape[2]
    QT = new_ref.shape[1]
    q = new_ref[0]
    dx = q[:, 0:1] - xyzT_ref[0, 0:1, :]
    dy = q[:, 1:2] - xyzT_ref[0, 1:2, :]
    dz = q[:, 2:3] - xyzT_ref[0, 2:3, :]
    d2 = (dx * dx + dy * dy) + dz * dz
    mask = (d2 < RADIUS * RADIUS).astype(jnp.int32)
    lane = lax.broadcasted_iota(jnp.int32, (QT, N), 1)
    c = mask
    sh = 1
    while sh < N:
        rolled = jnp.roll(c, sh, axis=1)
        c = c + jnp.where(lane >= sh, rolled, 0)
        sh *= 2
    cols = []
    cnt0 = jnp.sum((c <= 0).astype(jnp.int32), axis=1, keepdims=True)
    pad = jnp.where(cnt0 >= N, 0, cnt0)
    cols.append(pad)
    for j in range(1, KNBR):
        cj = jnp.sum((c <= j).astype(jnp.int32), axis=1, keepdims=True)
        cols.append(jnp.where(cj >= N, pad, cj))
    idx_ref[0] = jnp.concatenate(cols, axis=1) + b * N


def _dense_kernel(x_ref, xyzT_ref, w1f_ref, w1x_ref, b1_ref, z_ref):
    zf = lax.dot_general(x_ref[0], w1f_ref[...], (((0,), (1,)), ((), ())),
                         preferred_element_type=jnp.float32)
    zx = lax.dot_general(xyzT_ref[0], w1x_ref[...], (((0,), (1,)), ((), ())),
                         preferred_element_type=jnp.float32)
    z_ref[0] = (zf + zx) + b1_ref[...]


def _gather_rows(table, gidx):
    R = gidx.shape[0]
    D = table.shape[1]
    info = plsc.get_sparse_core_info()
    NC, NS = info.num_cores, info.num_subcores
    NW = NC * NS
    per_w = R // NW
    CH = 128
    steps = per_w // CH
    mesh = plsc.VectorSubcoreMesh(core_axis_name="c", subcore_axis_name="s")

    @functools.partial(
        pl.kernel,
        mesh=mesh,
        out_type=jax.ShapeDtypeStruct((R, D), jnp.float32),
        scratch_types=[
            pltpu.VMEM((CH,), jnp.int32),
            pltpu.VMEM((CH, D), jnp.float32),
            pltpu.SemaphoreType.DMA,
        ],
    )
    def gk(table_hbm, gidx_hbm, out_hbm, idx_v, rows_v, sem):
        wid = lax.axis_index("s") * NC + lax.axis_index("c")
        base = wid * per_w

        def step(i, carry):
            off = base + i * CH
            pltpu.sync_copy(gidx_hbm.at[pl.ds(off, CH)], idx_v)
            pltpu.async_copy(table_hbm.at[idx_v], rows_v, sem).wait()
            pltpu.sync_copy(rows_v, out_hbm.at[pl.ds(off, CH)])
            return carry

        lax.fori_loop(0, steps, step, 0)

    return gk(table, gidx)


def _mlp_kernel(g_ref, new_ref, w2_ref, b2_ref, w1x_ref, out_ref):
    QT, K, C1 = g_ref.shape
    C2 = out_ref.shape[1]
    q = new_ref[0]
    v = lax.dot_general(q, w1x_ref[...], (((1,), (1,)), ((), ())),
                        preferred_element_type=jnp.float32)
    h = jnp.maximum(g_ref[...] - v[:, None, :], 0.0)
    o = lax.dot_general(h.reshape(QT * K, C1), w2_ref[...],
                        (((1,), (1,)), ((), ())),
                        preferred_element_type=jnp.float32)
    o = o + b2_ref[...]
    out_ref[...] = jnp.max(o.reshape(QT, K, C2), axis=1)


def _bn_kernel(p_ref, gamma_ref, beta_ref, out_ref):
    p = p_ref[...]
    n = p.shape[0]
    mean = jnp.sum(p, axis=0, keepdims=True) / n
    d = p - mean
    var = jnp.sum(d * d, axis=0, keepdims=True) / n
    out_ref[...] = d / jnp.sqrt(var + EPS) * gamma_ref[...] + beta_ref[...]


def kernel(xyz, x, W1, b1, W2, b2, gamma, beta):
    B, N, _ = xyz.shape
    C0 = x.shape[1]
    C1 = W1.shape[0]
    C2 = W2.shape[0]
    S = N // STRIDE
    QT = 128
    QT2 = 128

    xyzT = jnp.transpose(xyz, (0, 2, 1))
    xyz_cl = xyzT.reshape(B, 3 * _SUB, N // _SUB)
    W1x = W1[:, :3]
    W1f = W1[:, 3:]
    b1r = b1.reshape(1, C1)
    b2r = b2.reshape(1, C2)
    gammar = gamma.reshape(1, C2)
    betar = beta.reshape(1, C2)

    new_xyz = pl.pallas_call(
        _fps_kernel,
        in_specs=[
            pl.BlockSpec((B, 3 * _SUB, N // _SUB), lambda: (0, 0, 0)),
            pl.BlockSpec((B, N, 3), lambda: (0, 0, 0)),
        ],
        out_specs=pl.BlockSpec((B, S, 3), lambda: (0, 0, 0)),
        out_shape=jax.ShapeDtypeStruct((B, S, 3), jnp.float32),
    )(xyz_cl, xyz)

    gidx = pl.pallas_call(
        _ballq_kernel,
        grid=(B, S // QT),
        in_specs=[
            pl.BlockSpec((1, 3, N), lambda b, t: (b, 0, 0)),
            pl.BlockSpec((1, QT, 3), lambda b, t: (b, t, 0)),
        ],
        out_specs=pl.BlockSpec((1, QT, KNBR), lambda b, t: (b, t, 0)),
        out_shape=jax.ShapeDtypeStruct((B, S, KNBR), jnp.int32),
    )(xyzT, new_xyz)

    z = pl.pallas_call(
        _dense_kernel,
        grid=(B,),
        in_specs=[
            pl.BlockSpec((1, C0, N), lambda b: (b, 0, 0)),
            pl.BlockSpec((1, 3, N), lambda b: (b, 0, 0)),
            pl.BlockSpec((C1, C0), lambda b: (0, 0)),
            pl.BlockSpec((C1, 3), lambda b: (0, 0)),
            pl.BlockSpec((1, C1), lambda b: (0, 0)),
        ],
        out_specs=pl.BlockSpec((1, N, C1), lambda b: (b, 0, 0)),
        out_shape=jax.ShapeDtypeStruct((B, N, C1), jnp.float32),
    )(x, xyzT, W1f, W1x, b1r)

    g = _gather_rows(z.reshape(B * N, C1), gidx.reshape(B * S * KNBR))
    g3 = g.reshape(B * S, KNBR, C1)

    nt = S // QT2
    pooled = pl.pallas_call(
        _mlp_kernel,
        grid=(B, nt),
        in_specs=[
            pl.BlockSpec((QT2, KNBR, C1), lambda b, t: (b * nt + t, 0, 0)),
            pl.BlockSpec((1, QT2, 3), lambda b, t: (b, t, 0)),
            pl.BlockSpec((C2, C1), lambda b, t: (0, 0)),
            pl.BlockSpec((1, C2), lambda b, t: (0, 0)),
            pl.BlockSpec((C1, 3), lambda b, t: (0, 0)),
        ],
        out_specs=pl.BlockSpec((QT2, C2), lambda b, t: (b * nt + t, 0)),
        out_shape=jax.ShapeDtypeStruct((B * S, C2), jnp.float32),
    )(g3, new_xyz, W2, b2r, W1x)

    bnout = pl.pallas_call(
        _bn_kernel,
        in_specs=[
            pl.BlockSpec((B * S, C2), lambda: (0, 0)),
            pl.BlockSpec((1, C2), lambda: (0, 0)),
            pl.BlockSpec((1, C2), lambda: (0, 0)),
        ],
        out_specs=pl.BlockSpec((B * S, C2), lambda: (0, 0)),
        out_shape=jax.ShapeDtypeStruct((B * S, C2), jnp.float32),
    )(pooled, gammar, betar)

    feat = jnp.transpose(bnout.reshape(B, S, C2), (0, 2, 1))
    return (new_xyz, feat)
